# baseline (speedup 1.0000x reference)
import dataclasses
import functools

import jax
import jax.numpy as jnp
import numpy as np
from jax import lax
from jax.experimental import pallas as pl
from jax.experimental.pallas import tpu as pltpu
from jax.experimental.pallas import tpu_sc as plsc

NC = 2
NS = 16
L = 16
NB = 4


def _sc_segment_sum(x_pk, src, dst3, w, n_rows):
    nw = NC * NS
    e_total = src.shape[0]
    d = 2 * x_pk.shape[1]
    epw = e_total // nw
    chunk = 80
    n_chunks = epw // chunk
    assert epw % chunk == 0 and dst3.shape == (nw, n_chunks, chunk)
    n_ep = 2 + (n_chunks - 2) % NB
    n_trips = (n_chunks - n_ep) // NB
    assert n_trips * NB + n_ep == n_chunks and n_ep <= NB + 1
    tile_step = (n_rows // NS) // 8 * 8
    tile_win = n_rows - tile_step * (NS - 1)
    assert tile_win % chunk == 0 and tile_win >= tile_step

    mesh = plsc.VectorSubcoreMesh(core_axis_name="c", subcore_axis_name="s")
    cp = pltpu.CompilerParams()
    if "needs_layout_passes" in pltpu.CompilerParams.__dataclass_fields__:
        cp = dataclasses.replace(cp, needs_layout_passes=False)
    if "use_tc_tiling_on_sc" in pltpu.CompilerParams.__dataclass_fields__:
        cp = dataclasses.replace(cp, use_tc_tiling_on_sc=False)

    @functools.partial(
        pl.kernel,
        out_type=jax.ShapeDtypeStruct((NC, n_rows, d), jnp.float32),
        mesh=mesh,
        compiler_params=cp,
        scratch_types=[
            pltpu.VMEM((n_chunks, chunk), jnp.int32),
            pltpu.VMEM((NB, chunk), jnp.int32),
            pltpu.VMEM((NB, chunk), jnp.float32),
            pltpu.VMEM((chunk, d // 2), jnp.int32),
            pltpu.VMEM((chunk, d // 2), jnp.int32),
            pltpu.VMEM((chunk, d), jnp.float32),
            pltpu.VMEM((chunk, d), jnp.float32),
            pltpu.VMEM_SHARED((n_rows, d), jnp.float32),
        ] + [pltpu.SemaphoreType.DMA] * (2 + 2 + NB + NB),
    )
    def sc_kernel(x_hbm, src_hbm, dst3_hbm, w_hbm, part_hbm,
                  di_all, si_r, w_r, rb0, rb1, st0, st1, acc_sh, *sems):
        sg = sems[0:2]
        ss = sems[2:4]
        sl = sems[4:4 + NB]
        sw = sems[4 + NB:]
        rbs = (rb0, rb1)
        sts = (st0, st1)
        cidx = lax.axis_index("c")
        sidx = lax.axis_index("s")
        wid = sidx * NC + cidx
        base = wid * epw

        cdi = pltpu.async_copy(dst3_hbm.at[wid], di_all, sg[0])

        zero = jnp.zeros((L,), jnp.float32)

        @pl.loop(0, chunk)
        def _(i):
            for j in range(d // L):
                st0[i, pl.ds(j * L, L)] = zero

        row0 = sidx * tile_step
        for k in range(tile_win // chunk):
            pltpu.sync_copy(st0, acc_sh.at[pl.ds(row0 + k * chunk, chunk)])
        cdi.wait()
        plsc.subcore_barrier()

        def si_start(k, b):
            pltpu.async_copy(
                src_hbm.at[pl.ds(base + k * chunk, chunk)], si_r.at[b], sl[b])

        def si_wait(k, b):
            pltpu.make_async_copy(
                src_hbm.at[pl.ds(base + k * chunk, chunk)], si_r.at[b],
                sl[b]).wait()

        def w_start(k, b):
            pltpu.async_copy(
                w_hbm.at[pl.ds(base + k * chunk, chunk)], w_r.at[b], sw[b])

        def w_wait(k, b):
            pltpu.make_async_copy(
                w_hbm.at[pl.ds(base + k * chunk, chunk)], w_r.at[b],
                sw[b]).wait()

        def gather_start(bi, br):
            pltpu.async_copy(x_hbm.at[si_r.at[bi]], rbs[br], sg[br])

        def gather_wait(bi, br):
            pltpu.make_async_copy(x_hbm.at[si_r.at[bi]], rbs[br], sg[br]).wait()

        def scat_start(k, bs):
            pltpu.async_copy(sts[bs], acc_sh.at[di_all.at[k]], ss[bs], add=True)

        def scat_wait(k, bs):
            pltpu.make_async_copy(
                sts[bs], acc_sh.at[di_all.at[k]], ss[bs]).wait()

        def scale(k, bw, br, bs):
            rows_v = rbs[br]
            stg_v = sts[bs]

            @pl.loop(0, chunk, step=L)
            def _(g):
                w16 = w_r[bw, pl.ds(g, L)]
                for u in range(L):
                    wv = lax.gather(
                        w16, jnp.full((L, 1), u, jnp.int32),
                        lax.GatherDimensionNumbers(
                            offset_dims=(), collapsed_slice_dims=(0,),
                            start_index_map=(0,)),
                        (1,), mode=lax.GatherScatterMode.PROMISE_IN_BOUNDS)
                    for j in range(d // (2 * L)):
                        ab32 = rows_v[g + u, pl.ds(L * j, L)]
                        a = plsc.bitcast(ab32 << 16, jnp.float32)
                        b = plsc.bitcast(
                            ab32 & jnp.int32(-65536), jnp.float32)
                        stg_v[g + u, pl.ds(2 * L * j, L)] = a * wv
                        stg_v[g + u, pl.ds(2 * L * j + L, L)] = b * wv

        def body(k, j, issue_gather, issue_load, wait_scat):
            br = j % 2
            bs = j % 2
            gather_wait(j, br)
            w_wait(k, j)
            if wait_scat:
                scat_wait(k - 2, bs)
            scale(k, j, br, bs)
            scat_start(k, bs)
            if issue_gather:
                b2 = (j + 2) % NB
                si_wait(k + 2, b2)
                gather_start(b2, br)
            if issue_load:
                b3 = (j + 3) % NB
                si_start(k + 3, b3)
                w_start(k + 3, b3)

        for k in range(3):
            si_start(k, k)
            w_start(k, k)
        si_wait(0, 0)
        gather_start(0, 0)
        si_wait(1, 1)
        gather_start(1, 1)

        @pl.loop(0, n_trips)
        def _(p):
            k0 = NB * p
            for j in range(NB):
                if j < 2:
                    @pl.when(k0 + j > 1)
                    def _(j=j):
                        scat_wait(k0 + j - 2, j % 2)
                    body(k0 + j, j, True, True, wait_scat=False)
                else:
                    body(k0 + j, j, True, True, wait_scat=True)

        ke = n_trips * NB
        for i in range(n_ep):
            k = ke + i
            body(k, k % NB, issue_gather=(k + 2 < n_chunks),
                 issue_load=(k + 3 < n_chunks), wait_scat=True)
        scat_wait(n_chunks - 2, (n_chunks - 2) % 2)
        scat_wait(n_chunks - 1, (n_chunks - 1) % 2)

        plsc.subcore_barrier()
        pltpu.sync_copy(acc_sh.at[pl.ds(row0, tile_win)],
                        part_hbm.at[cidx, pl.ds(row0, tile_win)])

    return sc_kernel(x_pk, src, dst3, w)


def _feature_perm(d):
    p = []
    for blk in range(0, d, 2 * L):
        p.extend(range(blk, blk + 2 * L, 2))
        p.extend(range(blk + 1, blk + 2 * L, 2))
    return np.array(p, dtype=np.int32)


def _tc_combine_matmul_relu(partials, W_perm):
    n_rows, d_in = partials.shape[1], partials.shape[2]
    d_out = W_perm.shape[1]
    blk = 1000

    def body(p_ref, w_ref, o_ref):
        p = p_ref[0] + p_ref[1]
        o_ref[...] = jnp.maximum(
            jnp.dot(p, w_ref[...], preferred_element_type=jnp.float32), 0.0)

    return pl.pallas_call(
        body,
        grid=(n_rows // blk,),
        in_specs=[
            pl.BlockSpec((NC, blk, d_in), lambda i: (0, i, 0)),
            pl.BlockSpec((d_in, d_out), lambda i: (0, 0)),
        ],
        out_specs=pl.BlockSpec((blk, d_out), lambda i: (i, 0)),
        out_shape=jax.ShapeDtypeStruct((n_rows, d_out), jnp.float32),
    )(partials, W_perm)


def kernel(x, edge_index, edge_weight, W):
    n_rows, d = x.shape
    nw = NC * NS
    epw = edge_index.shape[1] // nw
    chunk = 80
    dst3 = edge_index[0].reshape(nw, epw // chunk, chunk)
    src = edge_index[1]
    x_bf = x.astype(jnp.bfloat16)
    x_pk = lax.bitcast_convert_type(
        x_bf.reshape(n_rows, d // 2, 2), jnp.int32)
    W_perm = W[_feature_perm(d)]
    partials = _sc_segment_sum(x_pk, src, dst3, edge_weight, n_rows)
    return _tc_combine_matmul_relu(partials, W_perm)


# speedup vs baseline: 2.0161x; 2.0161x over previous
import dataclasses
import functools

import jax
import jax.numpy as jnp
from jax import lax
from jax.experimental import pallas as pl
from jax.experimental.pallas import tpu as pltpu
from jax.experimental.pallas import tpu_sc as plsc

NC = 2
NS = 16
L = 16
NB = 3


def _sc_segment_sum(x, src, dst3, w, n_rows):
    nw = NC * NS
    e_total = src.shape[0]
    d = x.shape[1]
    epw = e_total // nw
    chunk = 80
    n_chunks = epw // chunk
    assert epw % chunk == 0 and dst3.shape == (nw, n_chunks, chunk)
    n_trips = (n_chunks - 2) // NB
    assert n_trips * NB + 2 == n_chunks
    tile_step = (n_rows // NS) // 8 * 8
    tile_win = n_rows - tile_step * (NS - 1)
    assert tile_win % chunk == 0 and tile_win >= tile_step

    mesh = plsc.VectorSubcoreMesh(core_axis_name="c", subcore_axis_name="s")
    cp = pltpu.CompilerParams()
    if "needs_layout_passes" in pltpu.CompilerParams.__dataclass_fields__:
        cp = dataclasses.replace(cp, needs_layout_passes=False)

    @functools.partial(
        pl.kernel,
        out_type=jax.ShapeDtypeStruct((NC, n_rows, d), jnp.float32),
        mesh=mesh,
        compiler_params=cp,
        scratch_types=[
            pltpu.VMEM((n_chunks, chunk), jnp.int32),
            pltpu.VMEM((NB, chunk), jnp.int32),
            pltpu.VMEM((NB, chunk), jnp.float32),
            pltpu.VMEM((chunk, d), jnp.float32),
            pltpu.VMEM((chunk, d), jnp.float32),
            pltpu.VMEM((chunk, d), jnp.float32),
            pltpu.VMEM_SHARED((n_rows, d), jnp.float32),
        ] + [pltpu.SemaphoreType.DMA] * 12,
    )
    def sc_kernel(x_hbm, src_hbm, dst3_hbm, w_hbm, part_hbm,
                  di_all, si_r, w_r, rows0, rows1, rows2, acc_sh, *sems):
        sg = sems[0:3]
        ss = sems[3:6]
        sl = sems[6:9]
        sw = sems[9:12]
        rows = (rows0, rows1, rows2)
        cidx = lax.axis_index("c")
        sidx = lax.axis_index("s")
        wid = sidx * NC + cidx
        base = wid * epw

        cdi = pltpu.async_copy(dst3_hbm.at[wid], di_all, sg[0])

        zero = jnp.zeros((L,), jnp.float32)

        @pl.loop(0, chunk)
        def _(i):
            for j in range(d // L):
                rows0[i, pl.ds(j * L, L)] = zero

        row0 = sidx * tile_step
        for k in range(tile_win // chunk):
            pltpu.sync_copy(rows0, acc_sh.at[pl.ds(row0 + k * chunk, chunk)])
        cdi.wait()
        plsc.subcore_barrier()

        def si_start(k, b):
            pltpu.async_copy(
                src_hbm.at[pl.ds(base + k * chunk, chunk)], si_r.at[b], sl[b])

        def si_wait(k, b):
            pltpu.make_async_copy(
                src_hbm.at[pl.ds(base + k * chunk, chunk)], si_r.at[b],
                sl[b]).wait()

        def w_start(k, b):
            pltpu.async_copy(
                w_hbm.at[pl.ds(base + k * chunk, chunk)], w_r.at[b], sw[b])

        def w_wait(k, b):
            pltpu.make_async_copy(
                w_hbm.at[pl.ds(base + k * chunk, chunk)], w_r.at[b],
                sw[b]).wait()

        def gather_start(b):
            pltpu.async_copy(x_hbm.at[si_r.at[b]], rows[b], sg[b])

        def gather_wait(b):
            pltpu.make_async_copy(x_hbm.at[si_r.at[b]], rows[b], sg[b]).wait()

        def scat_start(k, b):
            pltpu.async_copy(rows[b], acc_sh.at[di_all.at[k]], ss[b], add=True)

        def scat_wait(k, b):
            pltpu.make_async_copy(rows[b], acc_sh.at[di_all.at[k]], ss[b]).wait()

        def scale(k, b):
            rows_v = rows[b]

            @pl.loop(0, chunk, step=L)
            def _(g):
                w16 = w_r[b, pl.ds(g, L)]
                for u in range(L):
                    wv = lax.gather(
                        w16, jnp.full((L, 1), u, jnp.int32),
                        lax.GatherDimensionNumbers(
                            offset_dims=(), collapsed_slice_dims=(0,),
                            start_index_map=(0,)),
                        (1,), mode=lax.GatherScatterMode.PROMISE_IN_BOUNDS)
                    for j in range(d // L):
                        sl_ = pl.ds(j * L, L)
                        rows_v[g + u, sl_] = rows_v[g + u, sl_] * wv

        def body(k, b, first, issue_next):
            gather_wait(b)
            w_wait(k, b)
            scale(k, b)
            scat_start(k, b)
            if first:
                @pl.when(k > 0)
                def _():
                    scat_wait(k - 1, (b + NB - 1) % NB)
            else:
                scat_wait(k - 1, (b + NB - 1) % NB)
            if issue_next:
                b2 = (b + 2) % NB

                @pl.when(k + 2 < n_chunks)
                def _():
                    si_wait(k + 2, b2)
                    gather_start(b2)

                @pl.when(k + 3 < n_chunks)
                def _():
                    si_start(k + 3, b)
                    w_start(k + 3, b)

        si_start(0, 0)
        w_start(0, 0)
        si_start(1, 1)
        w_start(1, 1)
        si_wait(0, 0)
        gather_start(0)
        si_start(2, 2)
        w_start(2, 2)
        si_wait(1, 1)
        gather_start(1)

        @pl.loop(0, n_trips)
        def _(p):
            k0 = NB * p
            body(k0, 0, True, True)
            body(k0 + 1, 1, False, True)
            body(k0 + 2, 2, False, True)

        kl = n_chunks - 2
        body(kl, 0, False, False)
        body(kl + 1, 1, False, False)
        scat_wait(kl + 1, 1)

        plsc.subcore_barrier()
        pltpu.sync_copy(acc_sh.at[pl.ds(row0, tile_win)],
                        part_hbm.at[cidx, pl.ds(row0, tile_win)])

    return sc_kernel(x, src, dst3, w)


def _tc_combine_matmul_relu(partials, W):
    n_rows, d_in = partials.shape[1], partials.shape[2]
    d_out = W.shape[1]
    blk = 1000

    def body(p_ref, w_ref, o_ref):
        p = p_ref[0] + p_ref[1]
        o_ref[...] = jnp.maximum(
            jnp.dot(p, w_ref[...], preferred_element_type=jnp.float32), 0.0)

    return pl.pallas_call(
        body,
        grid=(n_rows // blk,),
        in_specs=[
            pl.BlockSpec((NC, blk, d_in), lambda i: (0, i, 0)),
            pl.BlockSpec((d_in, d_out), lambda i: (0, 0)),
        ],
        out_specs=pl.BlockSpec((blk, d_out), lambda i: (i, 0)),
        out_shape=jax.ShapeDtypeStruct((n_rows, d_out), jnp.float32),
    )(partials, W)


def kernel(x, edge_index, edge_weight, W):
    n_rows = x.shape[0]
    nw = NC * NS
    epw = edge_index.shape[1] // nw
    chunk = 80
    dst3 = edge_index[0].reshape(nw, epw // chunk, chunk)
    src = edge_index[1]
    partials = _sc_segment_sum(x, src, dst3, edge_weight, n_rows)
    return _tc_combine_matmul_relu(partials, W)


# speedup vs baseline: 2.0523x; 1.0180x over previous
import dataclasses
import functools

import jax
import jax.numpy as jnp
from jax import lax
from jax.experimental import pallas as pl
from jax.experimental.pallas import tpu as pltpu
from jax.experimental.pallas import tpu_sc as plsc

NC = 2
NS = 16
L = 16
NB = 3


def _sc_segment_sum(x, src, dst3, w, n_rows):
    nw = NC * NS
    e_total = src.shape[0]
    d = x.shape[1]
    epw = e_total // nw
    chunk = 80
    n_chunks = epw // chunk
    assert epw % chunk == 0 and dst3.shape == (nw, n_chunks, chunk)
    n_trips = (n_chunks - 2) // NB
    assert n_trips * NB + 2 == n_chunks
    tile_step = (n_rows // NS) // 8 * 8
    tile_win = n_rows - tile_step * (NS - 1)
    assert tile_win % chunk == 0 and tile_win >= tile_step

    mesh = plsc.VectorSubcoreMesh(core_axis_name="c", subcore_axis_name="s")
    cp = pltpu.CompilerParams()
    if "needs_layout_passes" in pltpu.CompilerParams.__dataclass_fields__:
        cp = dataclasses.replace(cp, needs_layout_passes=False)

    @functools.partial(
        pl.kernel,
        out_type=jax.ShapeDtypeStruct((NC, n_rows, d), jnp.float32),
        mesh=mesh,
        compiler_params=cp,
        scratch_types=[
            pltpu.VMEM((n_chunks, chunk), jnp.int32),
            pltpu.VMEM((NB, chunk), jnp.int32),
            pltpu.VMEM((NB, chunk), jnp.float32),
            pltpu.VMEM((chunk, d), jnp.float32),
            pltpu.VMEM((chunk, d), jnp.float32),
            pltpu.VMEM((chunk, d), jnp.float32),
            pltpu.VMEM_SHARED((n_rows, d), jnp.float32),
        ] + [pltpu.SemaphoreType.DMA] * 12,
    )
    def sc_kernel(x_hbm, src_hbm, dst3_hbm, w_hbm, part_hbm,
                  di_all, si_r, w_r, rows0, rows1, rows2, acc_sh, *sems):
        sg = sems[0:3]
        ss = sems[3:6]
        sl = sems[6:9]
        sw = sems[9:12]
        rows = (rows0, rows1, rows2)
        cidx = lax.axis_index("c")
        sidx = lax.axis_index("s")
        wid = sidx * NC + cidx
        base = wid * epw

        cdi = pltpu.async_copy(dst3_hbm.at[wid], di_all, sg[0])

        zero = jnp.zeros((L,), jnp.float32)

        @pl.loop(0, chunk)
        def _(i):
            for j in range(d // L):
                rows0[i, pl.ds(j * L, L)] = zero

        row0 = sidx * tile_step
        for k in range(tile_win // chunk):
            pltpu.sync_copy(rows0, acc_sh.at[pl.ds(row0 + k * chunk, chunk)])
        cdi.wait()
        plsc.subcore_barrier()

        def si_start(k, b):
            pltpu.async_copy(
                src_hbm.at[pl.ds(base + k * chunk, chunk)], si_r.at[b], sl[b])

        def si_wait(k, b):
            pltpu.make_async_copy(
                src_hbm.at[pl.ds(base + k * chunk, chunk)], si_r.at[b],
                sl[b]).wait()

        def w_start(k, b):
            pltpu.async_copy(
                w_hbm.at[pl.ds(base + k * chunk, chunk)], w_r.at[b], sw[b])

        def w_wait(k, b):
            pltpu.make_async_copy(
                w_hbm.at[pl.ds(base + k * chunk, chunk)], w_r.at[b],
                sw[b]).wait()

        def gather_start(b):
            pltpu.async_copy(x_hbm.at[si_r.at[b]], rows[b], sg[b])

        def gather_wait(b):
            pltpu.make_async_copy(x_hbm.at[si_r.at[b]], rows[b], sg[b]).wait()

        def scat_start(k, b):
            pltpu.async_copy(rows[b], acc_sh.at[di_all.at[k]], ss[b], add=True)

        def scat_wait(k, b):
            pltpu.make_async_copy(rows[b], acc_sh.at[di_all.at[k]], ss[b]).wait()

        def scale(k, b):
            rows_v = rows[b]

            @pl.loop(0, chunk, step=L)
            def _(g):
                w16 = w_r[b, pl.ds(g, L)]
                for u in range(L):
                    wv = lax.gather(
                        w16, jnp.full((L, 1), u, jnp.int32),
                        lax.GatherDimensionNumbers(
                            offset_dims=(), collapsed_slice_dims=(0,),
                            start_index_map=(0,)),
                        (1,), mode=lax.GatherScatterMode.PROMISE_IN_BOUNDS)
                    for j in range(d // L):
                        sl_ = pl.ds(j * L, L)
                        rows_v[g + u, sl_] = rows_v[g + u, sl_] * wv

        def body(k, b, first, issue_next):
            gather_wait(b)
            w_wait(k, b)
            scale(k, b)
            scat_start(k, b)
            if first:
                @pl.when(k > 0)
                def _():
                    scat_wait(k - 1, (b + NB - 1) % NB)
            else:
                scat_wait(k - 1, (b + NB - 1) % NB)
            if issue_next:
                b2 = (b + 2) % NB

                @pl.when(k + 2 < n_chunks)
                def _():
                    si_wait(k + 2, b2)
                    gather_start(b2)

                @pl.when(k + 3 < n_chunks)
                def _():
                    si_start(k + 3, b)
                    w_start(k + 3, b)

        si_start(0, 0)
        w_start(0, 0)
        si_start(1, 1)
        w_start(1, 1)
        si_wait(0, 0)
        gather_start(0)
        si_start(2, 2)
        w_start(2, 2)
        si_wait(1, 1)
        gather_start(1)

        @pl.loop(0, n_trips)
        def _(p):
            k0 = NB * p
            body(k0, 0, True, True)
            body(k0 + 1, 1, False, True)
            body(k0 + 2, 2, False, True)

        kl = n_chunks - 2
        body(kl, 0, False, False)
        body(kl + 1, 1, False, False)
        scat_wait(kl + 1, 1)

        plsc.subcore_barrier()
        pltpu.sync_copy(acc_sh.at[pl.ds(row0, tile_win)],
                        part_hbm.at[cidx, pl.ds(row0, tile_win)])

    return sc_kernel(x, src, dst3, w)


def _tc_combine_matmul_relu(partials, W):
    n_rows, d_in = partials.shape[1], partials.shape[2]
    d_out = W.shape[1]
    blk = 2000

    def body(p_ref, w_ref, o_ref):
        p = p_ref[0] + p_ref[1]
        o_ref[...] = jnp.maximum(
            jnp.dot(p, w_ref[...], preferred_element_type=jnp.float32), 0.0)

    return pl.pallas_call(
        body,
        grid=(n_rows // blk,),
        in_specs=[
            pl.BlockSpec((NC, blk, d_in), lambda i: (0, i, 0)),
            pl.BlockSpec((d_in, d_out), lambda i: (0, 0)),
        ],
        out_specs=pl.BlockSpec((blk, d_out), lambda i: (i, 0)),
        out_shape=jax.ShapeDtypeStruct((n_rows, d_out), jnp.float32),
    )(partials, W)


def kernel(x, edge_index, edge_weight, W):
    n_rows = x.shape[0]
    nw = NC * NS
    epw = edge_index.shape[1] // nw
    chunk = 80
    dst3 = edge_index[0].reshape(nw, epw // chunk, chunk)
    src = edge_index[1]
    partials = _sc_segment_sum(x, src, dst3, edge_weight, n_rows)
    return _tc_combine_matmul_relu(partials, W)
